# baseline (speedup 1.0000x reference)
import jax, jax.numpy as jnp
from jax.experimental import pallas as pl

H = 64
C_DIM = 32
K = 16
EPS = 1e-8


def _instance_norm(x, w, b):
    m = jnp.mean(x, axis=0, keepdims=True)
    v = jnp.var(x, axis=0, keepdims=True)
    return (x - m) / jnp.sqrt(v + 1e-5) * w + b


def _silu(x):
    return jax.nn.silu(x)


def _knn_edges(p, tag, k):
    N = p.shape[0]
    sq = jnp.sum(p * p, axis=-1)
    d2 = sq[:, None] + sq[None, :] - 2.0 * (p @ p.T)
    bad = (tag[:, None] != tag[None, :]) | jnp.eye(N, dtype=bool)
    d2 = jnp.where(bad, jnp.inf, d2)
    _, idx = jax.lax.top_k(-d2, k)
    snd = idx.reshape(-1)
    rcv = jnp.repeat(jnp.arange(N), k)
    return snd, rcv


def _e_gcl(h, vec, edge_attr, snd, rcv, p):
    vec_diff = vec[snd] - vec[rcv]
    radial = jnp.sum(vec_diff ** 2, axis=1, keepdims=True)
    vec_diff = vec_diff / jax.lax.stop_gradient(jnp.sqrt(radial)) + EPS
    inp = jnp.concatenate([h[snd], h[rcv], radial, edge_attr], axis=1)
    m = _silu(_instance_norm(inp @ p['e_W1'] + p['e_b1'], p['e_nw'], p['e_nb']))
    edge_feat = _silu(m @ p['e_W2'] + p['e_b2'])
    vscale = _silu(_instance_norm(edge_feat @ p['v_W1'] + p['v_b1'], p['v_nw'], p['v_nb'])) @ p['v_W2']
    trans = vec_diff * vscale
    vec = vec + jnp.zeros_like(vec).at[snd].add(trans)
    agg = jnp.zeros((h.shape[0], edge_feat.shape[1]), jnp.float32).at[snd].add(edge_feat)
    nm = _silu(_instance_norm(jnp.concatenate([h, agg], axis=1) @ p['n_W1'] + p['n_b1'], p['n_nw'], p['n_nb']))
    h = h + (nm @ p['n_W2'] + p['n_b2'])
    return h, vec


def _proj_kernel(h_ref, w_ref, b_ref, o_ref):
    o_ref[...] = h_ref[...] @ w_ref[...] + b_ref[...]


def kernel(pc, node_tag, params):
    snd, rcv = _knn_edges(pc.reshape(-1, 3), node_tag.reshape(-1), K)
    bs, n_nodes, _ = pc.shape
    p = pc.reshape(-1, 3)
    tag = node_tag.reshape(-1)
    n_seg = 16
    counts = jax.ops.segment_sum(jnp.ones_like(tag, dtype=jnp.float32), tag, num_segments=n_seg)
    bary = jax.ops.segment_sum(p, tag, num_segments=n_seg) / counts[:, None]
    displ = p[snd] - p[rcv]
    distance = jnp.linalg.norm(displ, axis=-1, keepdims=True)
    direction = displ / (distance + EPS)
    x = p - bary[tag]
    loc = jnp.linalg.norm(p, axis=-1, keepdims=True)
    N = p.shape[0]
    dsum = jax.ops.segment_sum(distance[:, 0], snd, num_segments=N)
    dcnt = jax.ops.segment_sum(jnp.ones_like(distance[:, 0]), snd, num_segments=N)
    density = jnp.where(dcnt > 0, dsum / jnp.maximum(dcnt, 1.0), 0.0)[:, None]
    h = jnp.concatenate([loc, density], axis=-1)
    a = jnp.sum(direction[snd] * direction[rcv], axis=-1, keepdims=True)
    angles = jnp.arccos(jnp.clip(a, -1.0 + EPS, 1.0 - EPS))
    edge_attr = jnp.concatenate([distance, angles], axis=-1)
    h = h @ params['emb_W'] + params['emb_b']
    vec = x
    for lp in params['layers']:
        h, vec = _e_gcl(h, vec, edge_attr, snd, rcv, lp)
    s_codes = pl.pallas_call(
        _proj_kernel,
        out_shape=jax.ShapeDtypeStruct((N, C_DIM), jnp.float32),
    )(h, params['ro_W'], params['ro_b'][None, :]).reshape(bs, n_nodes, C_DIM)
    v_codes = vec.reshape(bs, n_nodes, 3)
    return v_codes, s_codes


# speedup vs baseline: 5.8023x; 5.8023x over previous
import functools

import jax
import jax.numpy as jnp
from jax import lax
from jax.experimental import pallas as pl
from jax.experimental.pallas import tpu as pltpu
from jax.experimental.pallas import tpu_sc as plsc

N = 4096
KNN = 16
H = 64
C_DIM = 32
E = N * KNN
EPS = 1e-8
NTAG = 16

NC, NS, L = 2, 16, 16
NW = NC * NS
QPT = N // NW
CPW = E // NW // 128
NB = 32
EB = E // NB
QB = N // NB
F32 = jnp.float32
I32 = jnp.int32
INF = float("inf")


def _mesh():
    return plsc.VectorSubcoreMesh(core_axis_name="c", subcore_axis_name="s",
                                  num_cores=NC, num_subcores=NS)


_SC_PARAMS = pltpu.CompilerParams(needs_layout_passes=False,
                                  use_tc_tiling_on_sc=False)


def _wid():
    return lax.axis_index("s") * NC + lax.axis_index("c")


def _bf16r(v):
    y = plsc.bitcast(v, I32)
    r = y + 0x7FFF + lax.bitwise_and(lax.shift_right_logical(y, 16), 1)
    r = lax.bitwise_and(r, jnp.int32(-65536))
    return plsc.bitcast(r, F32)


def _knn_body(px_h, py_h, pz_h, st_h, en_h,
              nbr_h, dxo_h, dyo_h, dzo_h,
              xs, ys, zs, stv, env, nbr_v, dxv, dyv, dzv):
    base = _wid() * QPT
    pltpu.sync_copy(px_h, xs)
    pltpu.sync_copy(py_h, ys)
    pltpu.sync_copy(pz_h, zs)
    pltpu.sync_copy(st_h, stv)
    pltpu.sync_copy(en_h, env)
    lanes = lax.broadcasted_iota(I32, (L,), 0)

    def per_query(j, carry):
        q = base + j
        qspl = jnp.full((L,), q, I32)
        st = jnp.max(plsc.load_gather(stv, [qspl])).astype(I32)
        en = jnp.max(plsc.load_gather(env, [qspl])).astype(I32)
        xq = plsc.load_gather(xs, [qspl])
        yq = plsc.load_gather(ys, [qspl])
        zq = plsc.load_gather(zs, [qspl])
        sqq = xq * xq + yq * yq + zq * zq
        xqb = _bf16r(xq)
        yqb = _bf16r(yq)
        zqb = _bf16r(zq)
        nchunk = (en - st + (L - 1)) // L

        def chunk_body(t, cur):
            cur_d, cur_i = cur
            idx = st + t * L + lanes
            valid = idx < en
            idxc = jnp.where(valid, idx, 0)
            xc = plsc.load_gather(xs, [idxc])
            yc = plsc.load_gather(ys, [idxc])
            zc = plsc.load_gather(zs, [idxc])
            sqc = xc * xc + yc * yc + zc * zc
            dot = (xqb * _bf16r(xc) + yqb * _bf16r(yc) + zqb * _bf16r(zc))
            d2 = (sqq + sqc) - 2.0 * dot
            bad = jnp.logical_or(jnp.logical_not(valid), idx == q)
            d2 = jnp.where(bad, INF, d2)
            better = jnp.any(d2 < jnp.max(cur_d))

            def merge(args):
                cd, ci, rd, ri = args
                rd_s, ri_s = plsc.sort_key_val(rd, ri)
                rrd = lax.rev(rd_s, (0,))
                rri = lax.rev(ri_s, (0,))
                m = cd <= rrd
                lo_d = jnp.where(m, cd, rrd)
                lo_i = jnp.where(m, ci, rri)
                return tuple(plsc.sort_key_val(lo_d, lo_i))

            def keep(args):
                return args[0], args[1]

            return lax.cond(better, merge, keep, (cur_d, cur_i, d2, idx))

        cur_d0 = jnp.full((L,), INF, F32)
        cur_i0 = jnp.zeros((L,), I32)
        cur_d, cur_i = lax.fori_loop(0, nchunk, chunk_body, (cur_d0, cur_i0))

        nbr_v[j] = cur_i
        xn = plsc.load_gather(xs, [cur_i])
        yn = plsc.load_gather(ys, [cur_i])
        zn = plsc.load_gather(zs, [cur_i])
        dxv[j] = xn - xq
        dyv[j] = yn - yq
        dzv[j] = zn - zq
        return carry

    lax.fori_loop(0, QPT, per_query, 0)
    pltpu.sync_copy(nbr_v, nbr_h.at[pl.ds(base, QPT)])
    pltpu.sync_copy(dxv, dxo_h.at[pl.ds(base, QPT)])
    pltpu.sync_copy(dyv, dyo_h.at[pl.ds(base, QPT)])
    pltpu.sync_copy(dzv, dzo_h.at[pl.ds(base, QPT)])


def _knn_call(px, py, pz, st, en):
    f = pl.kernel(
        _knn_body,
        out_type=(
            jax.ShapeDtypeStruct((N, KNN), I32),
            jax.ShapeDtypeStruct((N, KNN), F32),
            jax.ShapeDtypeStruct((N, KNN), F32),
            jax.ShapeDtypeStruct((N, KNN), F32),
        ),
        mesh=_mesh(),
        compiler_params=_SC_PARAMS,
        scratch_types=[
            pltpu.VMEM((N,), F32), pltpu.VMEM((N,), F32), pltpu.VMEM((N,), F32),
            pltpu.VMEM((N,), F32), pltpu.VMEM((N,), F32),
            pltpu.VMEM((QPT, KNN), I32),
            pltpu.VMEM((QPT, KNN), F32), pltpu.VMEM((QPT, KNN), F32),
            pltpu.VMEM((QPT, KNN), F32),
        ],
    )
    return f(px, py, pz, st, en)


def _post_body(nbr_h, dist_h, d0x_h, d0y_h, d0z_h,
               densp_h, a_h,
               d0xv, d0yv, d0zv, nbr_v, dist_v, acc, av):
    wid = _wid()
    base = wid * QPT
    pltpu.sync_copy(d0x_h, d0xv)
    pltpu.sync_copy(d0y_h, d0yv)
    pltpu.sync_copy(d0z_h, d0zv)
    pltpu.sync_copy(nbr_h.at[pl.ds(base, QPT)], nbr_v)
    pltpu.sync_copy(dist_h.at[pl.ds(base, QPT)], dist_v)

    zero16 = jnp.zeros((L,), F32)
    zero16i = jnp.zeros((L,), I32)
    one16i = jnp.ones((L,), I32)
    one16f = jnp.ones((L,), F32)

    def zero_row(i, carry):
        acc[i] = zero16
        return carry
    lax.fori_loop(0, N, zero_row, 0)

    def per_query(j, carry):
        ii = nbr_v[j]
        d = dist_v[j]
        plsc.addupdate_scatter(acc, [ii, zero16i], d)
        plsc.addupdate_scatter(acc, [ii, one16i], one16f)
        gx = plsc.load_gather(d0xv, [ii])
        gy = plsc.load_gather(d0yv, [ii])
        gz = plsc.load_gather(d0zv, [ii])
        qspl = jnp.full((L,), base + j, I32)
        rx = plsc.load_gather(d0xv, [qspl])
        ry = plsc.load_gather(d0yv, [qspl])
        rz = plsc.load_gather(d0zv, [qspl])
        av[j] = gx * rx + gy * ry + gz * rz
        return carry

    lax.fori_loop(0, QPT, per_query, 0)
    pltpu.sync_copy(acc, densp_h.at[wid])
    pltpu.sync_copy(av, a_h.at[pl.ds(base, QPT)])


def _post_call(nbr, dist, d0x, d0y, d0z):
    f = pl.kernel(
        _post_body,
        out_type=(
            jax.ShapeDtypeStruct((NW, N, L), F32),
            jax.ShapeDtypeStruct((N, KNN), F32),
        ),
        mesh=_mesh(),
        compiler_params=_SC_PARAMS,
        scratch_types=[
            pltpu.VMEM((N,), F32), pltpu.VMEM((N,), F32), pltpu.VMEM((N,), F32),
            pltpu.VMEM((QPT, KNN), I32), pltpu.VMEM((QPT, KNN), F32),
            pltpu.VMEM((N, L), F32),
            pltpu.VMEM((QPT, KNN), F32),
        ],
    )
    return f(nbr, dist, d0x, d0y, d0z)


def _gather_body(snd_h, h_h, vec_h, hs_h, vs_h,
                 idxv, hrows, vrows, sem):
    wid = _wid()
    pltpu.sync_copy(snd_h.at[pl.ds(wid * CPW, CPW)], idxv)

    def chunk(t, carry):
        off = wid * (CPW * 128) + t * 128
        isl = idxv.at[t]
        pltpu.async_copy(h_h.at[isl], hrows, sem).wait()
        pltpu.sync_copy(hrows, hs_h.at[pl.ds(off, 128)])
        pltpu.async_copy(vec_h.at[isl], vrows, sem).wait()
        pltpu.sync_copy(vrows, vs_h.at[pl.ds(off, 128)])
        return carry

    lax.fori_loop(0, CPW, chunk, 0)


def _gather_call(snd2d, h, vec):
    f = pl.kernel(
        _gather_body,
        out_type=(
            jax.ShapeDtypeStruct((E, H), F32),
            jax.ShapeDtypeStruct((E, L), F32),
        ),
        mesh=_mesh(),
        compiler_params=_SC_PARAMS,
        scratch_types=[
            pltpu.VMEM((CPW, 128), I32),
            pltpu.VMEM((128, H), F32),
            pltpu.VMEM((128, L), F32),
            pltpu.SemaphoreType.DMA,
        ],
    )
    return f(snd2d, h, vec)


def _scatter_body(snd_h, ef_h, t_h, z64_h, z16_h,
                  aggp_h, vecp_h,
                  acca, accb, idxv, rows64, rows16):
    cid = lax.axis_index("c")
    sid = lax.axis_index("s")
    wid = sid * NC + cid

    @pl.when(sid == 0)
    def _():
        pltpu.sync_copy(z64_h, acca)
        pltpu.sync_copy(z16_h, accb)

    plsc.subcore_barrier()
    pltpu.sync_copy(snd_h.at[pl.ds(wid * CPW, CPW)], idxv)

    def chunk(t, carry):
        off = wid * (CPW * 128) + t * 128
        isl = idxv.at[t]
        pltpu.sync_copy(ef_h.at[pl.ds(off, 128)], rows64)
        pltpu.sync_copy(rows64, acca.at[isl], add=True)
        pltpu.sync_copy(t_h.at[pl.ds(off, 128)], rows16)
        pltpu.sync_copy(rows16, accb.at[isl], add=True)
        return carry

    lax.fori_loop(0, CPW, chunk, 0)
    plsc.subcore_barrier()
    rows = N // NS
    pltpu.sync_copy(acca.at[pl.ds(sid * rows, rows)],
                    aggp_h.at[cid, pl.ds(sid * rows, rows)])
    pltpu.sync_copy(accb.at[pl.ds(sid * rows, rows)],
                    vecp_h.at[cid, pl.ds(sid * rows, rows)])


def _scatter_call(snd2d, ef, t, z64, z16):
    f = pl.kernel(
        _scatter_body,
        out_type=(
            jax.ShapeDtypeStruct((NC, N, H), F32),
            jax.ShapeDtypeStruct((NC, N, L), F32),
        ),
        mesh=_mesh(),
        compiler_params=_SC_PARAMS,
        scratch_types=[
            pltpu.VMEM_SHARED((N, H), F32),
            pltpu.VMEM_SHARED((N, L), F32),
            pltpu.VMEM((CPW, 128), I32),
            pltpu.VMEM((128, H), F32),
            pltpu.VMEM((128, L), F32),
        ],
    )
    return f(snd2d, ef, t, z64, z16)


def _bdot(a, b):
    return jnp.dot(a.astype(jnp.bfloat16), b.astype(jnp.bfloat16),
                   preferred_element_type=F32)


def _b16(x):
    return x.astype(jnp.bfloat16).astype(F32)


def _geom_body(dx_ref, dy_ref, dz_ref, dist_ref, dirx_ref, diry_ref, dirz_ref):
    dx = dx_ref[...]
    dy = dy_ref[...]
    dz = dz_ref[...]
    dist = jnp.sqrt(dx * dx + dy * dy + dz * dz)
    dist_ref[...] = dist
    den = dist + EPS
    dirx_ref[...] = dx / den
    diry_ref[...] = dy / den
    dirz_ref[...] = dz / den


def _geom_call(dx, dy, dz):
    return pl.pallas_call(
        _geom_body,
        out_shape=tuple(jax.ShapeDtypeStruct((N, KNN), F32) for _ in range(4)),
    )(dx, dy, dz)


def _densred_body(densp_ref, o_ref):
    @pl.when(pl.program_id(0) == 0)
    def _():
        o_ref[...] = jnp.zeros((N, L), F32)

    o_ref[...] += densp_ref[0]


def _densred_call(densp):
    return pl.pallas_call(
        _densred_body,
        grid=(NW,),
        in_specs=[pl.BlockSpec((1, N, L), lambda b: (b, 0, 0))],
        out_specs=pl.BlockSpec((N, L), lambda b: (0, 0)),
        out_shape=jax.ShapeDtypeStruct((N, L), F32),
    )(densp)


def _nodesetup_body(p16_ref, oht_ref, oh_ref, dens_ref, a_ref, emb_ref,
                    h0_ref, vec0_ref, ang_ref):
    p16 = p16_ref[...]
    dens2 = dens_ref[...]
    dsum = dens2[:, 0:1]
    dcnt = dens2[:, 1:2]
    density = jnp.where(dcnt > 0, dsum / jnp.maximum(dcnt, 1.0), 0.0)
    li = lax.broadcasted_iota(I32, (N, L), 1)
    pm = jnp.where(li < 3, p16, 0.0)
    loc = jnp.sqrt(jnp.sum(pm * pm, axis=1, keepdims=True))
    emb = emb_ref[...]
    h0_ref[...] = (_b16(loc) * _b16(emb[0:1]) + _b16(density) * _b16(emb[1:2])
                   + emb[2:3])
    seg = jnp.dot(oht_ref[...], p16, preferred_element_type=F32)
    bary = seg / seg[:, 3:4]
    vec0_ref[...] = p16 - jnp.dot(oh_ref[...], bary, preferred_element_type=F32)
    a = jnp.clip(a_ref[...], -1.0 + EPS, 1.0 - EPS)
    ang_ref[...] = jnp.arctan2(jnp.sqrt(jnp.maximum(1.0 - a * a, 0.0)), a)


def _nodesetup_call(p16, oht, oh, dens, a, emb):
    return pl.pallas_call(
        _nodesetup_body,
        out_shape=(
            jax.ShapeDtypeStruct((N, H), F32),
            jax.ShapeDtypeStruct((N, L), F32),
            jax.ShapeDtypeStruct((N, KNN), F32),
        ),
    )(p16, oht, oh, dens, a, emb)


def _edge1_body(hs_ref, h_ref, vs_ref, vec_ref, ea_ref,
                w1a_ref, w1b_ref, w1c_ref, wm_ref,
                e1_ref, vd_ref, st_ref):
    hs = hs_ref[...]
    hr = jnp.reshape(
        jnp.broadcast_to(h_ref[...][:, None, :], (QB, KNN, H)), (EB, H))
    vr = jnp.reshape(
        jnp.broadcast_to(vec_ref[...][:, None, :], (QB, KNN, L)), (EB, L))
    vd = vs_ref[...] - vr
    radial = jnp.sum(vd * vd, axis=1, keepdims=True)
    vd_ref[...] = vd / jnp.sqrt(radial) + EPS
    wm = wm_ref[...]
    e1 = (_bdot(hs, w1a_ref[...])
          + _bdot(hr, w1b_ref[...])
          + _bdot(ea_ref[...], w1c_ref[...])
          + _b16(radial) * _b16(wm[0:1]) + wm[1:2])
    e1_ref[...] = e1

    @pl.when(pl.program_id(0) == 0)
    def _():
        st_ref[...] = jnp.zeros((8, H), F32)

    s1 = jnp.sum(e1, axis=0)
    s2 = jnp.sum(e1 * e1, axis=0)
    st_ref[...] += jnp.concatenate(
        [s1[None, :], s2[None, :], jnp.zeros((6, H), F32)], axis=0)


def _edge1_call(hs, h, vs, vec, ea, w1a, w1b, w1c, wm):
    return pl.pallas_call(
        _edge1_body,
        grid=(NB,),
        in_specs=[
            pl.BlockSpec((EB, H), lambda b: (b, 0)),
            pl.BlockSpec((QB, H), lambda b: (b, 0)),
            pl.BlockSpec((EB, L), lambda b: (b, 0)),
            pl.BlockSpec((QB, L), lambda b: (b, 0)),
            pl.BlockSpec((EB, 8), lambda b: (b, 0)),
            pl.BlockSpec((H, H), lambda b: (0, 0)),
            pl.BlockSpec((H, H), lambda b: (0, 0)),
            pl.BlockSpec((8, H), lambda b: (0, 0)),
            pl.BlockSpec((8, H), lambda b: (0, 0)),
        ],
        out_specs=(
            pl.BlockSpec((EB, H), lambda b: (b, 0)),
            pl.BlockSpec((EB, L), lambda b: (b, 0)),
            pl.BlockSpec((8, H), lambda b: (0, 0)),
        ),
        out_shape=(
            jax.ShapeDtypeStruct((E, H), F32),
            jax.ShapeDtypeStruct((E, L), F32),
            jax.ShapeDtypeStruct((8, H), F32),
        ),
    )(hs, h, vs, vec, ea, w1a, w1b, w1c, wm)


def _edge2_body(e1_ref, st1_ref, ew2_ref, vw1_ref, wm_ref,
                ef_ref, st_ref):
    st1 = st1_ref[...]
    mean = st1[0:1] / E
    var = st1[1:2] / E - mean * mean
    wm = wm_ref[...]
    x = (e1_ref[...] - mean) / jnp.sqrt(var + 1e-5) * wm[0:1] + wm[1:2]
    m = jax.nn.silu(x)
    ef = jax.nn.silu(_bdot(m, ew2_ref[...]) + wm[2:3])
    ef_ref[...] = ef
    v1 = _bdot(ef, vw1_ref[...]) + wm[3:4]

    @pl.when(pl.program_id(0) == 0)
    def _():
        st_ref[...] = jnp.zeros((8, H), F32)

    s1 = jnp.sum(v1, axis=0)
    s2 = jnp.sum(v1 * v1, axis=0)
    st_ref[...] += jnp.concatenate(
        [s1[None, :], s2[None, :], jnp.zeros((6, H), F32)], axis=0)


def _edge2_call(e1, st1, ew2, vw1, wm):
    return pl.pallas_call(
        _edge2_body,
        grid=(NB,),
        in_specs=[
            pl.BlockSpec((EB, H), lambda b: (b, 0)),
            pl.BlockSpec((8, H), lambda b: (0, 0)),
            pl.BlockSpec((H, H), lambda b: (0, 0)),
            pl.BlockSpec((H, H), lambda b: (0, 0)),
            pl.BlockSpec((8, H), lambda b: (0, 0)),
        ],
        out_specs=(
            pl.BlockSpec((EB, H), lambda b: (b, 0)),
            pl.BlockSpec((8, H), lambda b: (0, 0)),
        ),
        out_shape=(
            jax.ShapeDtypeStruct((E, H), F32),
            jax.ShapeDtypeStruct((8, H), F32),
        ),
    )(e1, st1, ew2, vw1, wm)


def _edge3_body(ef_ref, vd_ref, st2_ref, vw1_ref, vw2_ref, wm_ref, t_ref):
    st2 = st2_ref[...]
    mean = st2[0:1] / E
    var = st2[1:2] / E - mean * mean
    wm = wm_ref[...]
    v1 = _bdot(ef_ref[...], vw1_ref[...]) + wm[2:3]
    m2 = jax.nn.silu((v1 - mean) / jnp.sqrt(var + 1e-5) * wm[0:1] + wm[1:2])
    vs8 = _bdot(m2, vw2_ref[...])
    vscale = vs8[:, 0:1]
    t_ref[...] = vd_ref[...] * vscale


def _edge3_call(ef, vd, st2, vw1, vw2, wm):
    return pl.pallas_call(
        _edge3_body,
        grid=(NB,),
        in_specs=[
            pl.BlockSpec((EB, H), lambda b: (b, 0)),
            pl.BlockSpec((EB, L), lambda b: (b, 0)),
            pl.BlockSpec((8, H), lambda b: (0, 0)),
            pl.BlockSpec((H, H), lambda b: (0, 0)),
            pl.BlockSpec((H, 8), lambda b: (0, 0)),
            pl.BlockSpec((8, H), lambda b: (0, 0)),
        ],
        out_specs=pl.BlockSpec((EB, L), lambda b: (b, 0)),
        out_shape=jax.ShapeDtypeStruct((E, L), F32),
    )(ef, vd, st2, vw1, vw2, wm)


def _node_body(h_ref, aggp_ref, vec_ref, vecp_ref,
               nw1a_ref, nw1b_ref, nw2_ref, wm_ref,
               ho_ref, vo_ref):
    h = h_ref[...]
    agg = aggp_ref[0] + aggp_ref[1]
    wm = wm_ref[...]
    nm1 = (_bdot(h, nw1a_ref[...])
           + _bdot(agg, nw1b_ref[...])
           + wm[0:1])
    mean = jnp.mean(nm1, axis=0, keepdims=True)
    var = jnp.mean((nm1 - mean) ** 2, axis=0, keepdims=True)
    nm = jax.nn.silu((nm1 - mean) / jnp.sqrt(var + 1e-5) * wm[1:2] + wm[2:3])
    ho_ref[...] = h + _bdot(nm, nw2_ref[...]) + wm[3:4]
    vadd = vecp_ref[0] + vecp_ref[1]
    li = lax.broadcasted_iota(I32, (N, L), 1)
    vo_ref[...] = vec_ref[...] + jnp.where(li < 3, vadd, 0.0)


def _node_call(h, aggp, vec, vecp, nw1a, nw1b, nw2, wm):
    return pl.pallas_call(
        _node_body,
        out_shape=(
            jax.ShapeDtypeStruct((N, H), F32),
            jax.ShapeDtypeStruct((N, L), F32),
        ),
    )(h, aggp, vec, vecp, nw1a, nw1b, nw2, wm)


def _proj_body(h_ref, w_ref, b_ref, o_ref):
    o_ref[...] = _bdot(h_ref[...], w_ref[...]) + b_ref[0:1]


def _proj_call(h, w, b8):
    return pl.pallas_call(
        _proj_body,
        out_shape=jax.ShapeDtypeStruct((N, C_DIM), F32),
    )(h, w, b8)


def kernel(pc, node_tag, params):
    p = pc.reshape(N, 3).astype(F32)
    tag = node_tag.reshape(N).astype(I32)
    px, py, pz = p[:, 0], p[:, 1], p[:, 2]

    bounds = jnp.searchsorted(tag, jnp.arange(NTAG + 1, dtype=I32)).astype(I32)
    st = bounds[tag].astype(F32)
    en = bounds[tag + 1].astype(F32)

    nbr, dxm, dym, dzm = _knn_call(px, py, pz, st, en)

    dist, dirx, diry, dirz = _geom_call(dxm, dym, dzm)

    d0x = dirx[:N // KNN].reshape(N)
    d0y = diry[:N // KNN].reshape(N)
    d0z = dirz[:N // KNN].reshape(N)
    densp, a = _post_call(nbr, dist, d0x, d0y, d0z)

    p16 = jnp.concatenate(
        [p, jnp.ones((N, 1), F32), jnp.zeros((N, L - 4), F32)], axis=1)
    oh = (tag[:, None] == jnp.arange(NTAG, dtype=I32)[None, :]).astype(F32)
    oht = oh.T
    emb = jnp.concatenate([
        params['emb_W'],
        params['emb_b'][None, :],
        jnp.zeros((5, H), F32)], axis=0)
    dens = _densred_call(densp)
    h, vec, ang = _nodesetup_call(p16, oht, oh, dens, a, emb)

    ea8 = jnp.concatenate(
        [dist.reshape(E, 1), ang.reshape(E, 1), jnp.zeros((E, 6), F32)], axis=1)
    snd2d = nbr.reshape(E // 128, 128)
    z64 = jnp.zeros((N, H), F32)
    z16 = jnp.zeros((N, L), F32)

    for lp in params['layers']:
        w1a = lp['e_W1'][:H]
        w1b = lp['e_W1'][H:2 * H]
        w1c = jnp.concatenate(
            [lp['e_W1'][2 * H + 1:], jnp.zeros((6, H), F32)], axis=0)
        wm1 = jnp.concatenate([
            lp['e_W1'][2 * H:2 * H + 1],
            lp['e_b1'][None, :],
            jnp.zeros((6, H), F32)], axis=0)
        wm2 = jnp.concatenate([
            lp['e_nw'][None, :], lp['e_nb'][None, :],
            lp['e_b2'][None, :], lp['v_b1'][None, :],
            jnp.zeros((4, H), F32)], axis=0)
        wm3 = jnp.concatenate([
            lp['v_nw'][None, :], lp['v_nb'][None, :],
            lp['v_b1'][None, :],
            jnp.zeros((5, H), F32)], axis=0)
        vw2 = jnp.concatenate([lp['v_W2'], jnp.zeros((H, 7), F32)], axis=1)
        wmn = jnp.concatenate([
            lp['n_b1'][None, :], lp['n_nw'][None, :],
            lp['n_nb'][None, :], lp['n_b2'][None, :],
            jnp.zeros((4, H), F32)], axis=0)

        hs, vs = _gather_call(snd2d, h, vec)
        e1, vd, st1 = _edge1_call(hs, h, vs, vec, ea8, w1a, w1b, w1c, wm1)
        ef, st2 = _edge2_call(e1, st1, lp['e_W2'], lp['v_W1'], wm2)
        t = _edge3_call(ef, vd, st2, lp['v_W1'], vw2, wm3)
        aggp, vecp = _scatter_call(snd2d, ef, t, z64, z16)
        h, vec = _node_call(h, aggp, vec, vecp,
                            lp['n_W1'][:H], lp['n_W1'][H:], lp['n_W2'], wmn)

    rob = jnp.concatenate(
        [params['ro_b'][None, :], jnp.zeros((7, C_DIM), F32)], axis=0)
    s_codes = _proj_call(h, params['ro_W'], rob).reshape(1, N, C_DIM)
    v_codes = vec[:, :3].reshape(1, N, 3)
    return v_codes, s_codes


# speedup vs baseline: 6.0350x; 1.0401x over previous
import functools

import jax
import jax.numpy as jnp
from jax import lax
from jax.experimental import pallas as pl
from jax.experimental.pallas import tpu as pltpu
from jax.experimental.pallas import tpu_sc as plsc

N = 4096
KNN = 16
H = 64
C_DIM = 32
E = N * KNN
EPS = 1e-8
NTAG = 16

NC, NS, L = 2, 16, 16
NW = NC * NS
QPT = N // NW
CPW = E // NW // 128
NB = 32
EB = E // NB
QB = N // NB
F32 = jnp.float32
I32 = jnp.int32
INF = float("inf")


def _mesh():
    return plsc.VectorSubcoreMesh(core_axis_name="c", subcore_axis_name="s",
                                  num_cores=NC, num_subcores=NS)


_SC_PARAMS = pltpu.CompilerParams(needs_layout_passes=False,
                                  use_tc_tiling_on_sc=False)


def _wid():
    return lax.axis_index("s") * NC + lax.axis_index("c")


def _bf16r(v):
    y = plsc.bitcast(v, I32)
    r = y + 0x7FFF + lax.bitwise_and(lax.shift_right_logical(y, 16), 1)
    r = lax.bitwise_and(r, jnp.int32(-65536))
    return plsc.bitcast(r, F32)


def _knn_body(px_h, py_h, pz_h, st_h, en_h,
              nbr_h, dxo_h, dyo_h, dzo_h,
              xs, ys, zs, stv, env, nbr_v, dxv, dyv, dzv):
    base = _wid() * QPT
    pltpu.sync_copy(px_h, xs)
    pltpu.sync_copy(py_h, ys)
    pltpu.sync_copy(pz_h, zs)
    pltpu.sync_copy(st_h, stv)
    pltpu.sync_copy(en_h, env)
    lanes = lax.broadcasted_iota(I32, (L,), 0)

    def per_query(j, carry):
        q = base + j
        qspl = jnp.full((L,), q, I32)
        st = jnp.max(plsc.load_gather(stv, [qspl])).astype(I32)
        en = jnp.max(plsc.load_gather(env, [qspl])).astype(I32)
        xq = plsc.load_gather(xs, [qspl])
        yq = plsc.load_gather(ys, [qspl])
        zq = plsc.load_gather(zs, [qspl])
        sqq = xq * xq + yq * yq + zq * zq
        xqb = _bf16r(xq)
        yqb = _bf16r(yq)
        zqb = _bf16r(zq)
        nchunk = (en - st + (L - 1)) // L

        def chunk_body(t, cur):
            cur_d, cur_i = cur
            idx = st + t * L + lanes
            valid = idx < en
            idxc = jnp.where(valid, idx, 0)
            xc = plsc.load_gather(xs, [idxc])
            yc = plsc.load_gather(ys, [idxc])
            zc = plsc.load_gather(zs, [idxc])
            sqc = xc * xc + yc * yc + zc * zc
            dot = (xqb * _bf16r(xc) + yqb * _bf16r(yc) + zqb * _bf16r(zc))
            d2 = (sqq + sqc) - 2.0 * dot
            bad = jnp.logical_or(jnp.logical_not(valid), idx == q)
            d2 = jnp.where(bad, INF, d2)
            better = jnp.any(d2 < jnp.max(cur_d))

            def merge(args):
                cd, ci, rd, ri = args
                rd_s, ri_s = plsc.sort_key_val(rd, ri)
                rrd = lax.rev(rd_s, (0,))
                rri = lax.rev(ri_s, (0,))
                m = cd <= rrd
                lo_d = jnp.where(m, cd, rrd)
                lo_i = jnp.where(m, ci, rri)
                return tuple(plsc.sort_key_val(lo_d, lo_i))

            def keep(args):
                return args[0], args[1]

            return lax.cond(better, merge, keep, (cur_d, cur_i, d2, idx))

        cur_d0 = jnp.full((L,), INF, F32)
        cur_i0 = jnp.zeros((L,), I32)
        cur_d, cur_i = lax.fori_loop(0, nchunk, chunk_body, (cur_d0, cur_i0))

        nbr_v[j] = cur_i
        xn = plsc.load_gather(xs, [cur_i])
        yn = plsc.load_gather(ys, [cur_i])
        zn = plsc.load_gather(zs, [cur_i])
        dxv[j] = xn - xq
        dyv[j] = yn - yq
        dzv[j] = zn - zq
        return carry

    lax.fori_loop(0, QPT, per_query, 0)
    pltpu.sync_copy(nbr_v, nbr_h.at[pl.ds(base, QPT)])
    pltpu.sync_copy(dxv, dxo_h.at[pl.ds(base, QPT)])
    pltpu.sync_copy(dyv, dyo_h.at[pl.ds(base, QPT)])
    pltpu.sync_copy(dzv, dzo_h.at[pl.ds(base, QPT)])


def _knn_call(px, py, pz, st, en):
    f = pl.kernel(
        _knn_body,
        out_type=(
            jax.ShapeDtypeStruct((N, KNN), I32),
            jax.ShapeDtypeStruct((N, KNN), F32),
            jax.ShapeDtypeStruct((N, KNN), F32),
            jax.ShapeDtypeStruct((N, KNN), F32),
        ),
        mesh=_mesh(),
        compiler_params=_SC_PARAMS,
        scratch_types=[
            pltpu.VMEM((N,), F32), pltpu.VMEM((N,), F32), pltpu.VMEM((N,), F32),
            pltpu.VMEM((N,), F32), pltpu.VMEM((N,), F32),
            pltpu.VMEM((QPT, KNN), I32),
            pltpu.VMEM((QPT, KNN), F32), pltpu.VMEM((QPT, KNN), F32),
            pltpu.VMEM((QPT, KNN), F32),
        ],
    )
    return f(px, py, pz, st, en)


def _post_body(nbr_h, dist_h, d0x_h, d0y_h, d0z_h,
               densp_h, a_h,
               d0xv, d0yv, d0zv, nbr_v, dist_v, acc, av):
    wid = _wid()
    base = wid * QPT
    pltpu.sync_copy(d0x_h, d0xv)
    pltpu.sync_copy(d0y_h, d0yv)
    pltpu.sync_copy(d0z_h, d0zv)
    pltpu.sync_copy(nbr_h.at[pl.ds(base, QPT)], nbr_v)
    pltpu.sync_copy(dist_h.at[pl.ds(base, QPT)], dist_v)

    zero16 = jnp.zeros((L,), F32)
    zero16i = jnp.zeros((L,), I32)
    one16i = jnp.ones((L,), I32)
    one16f = jnp.ones((L,), F32)

    def zero_row(i, carry):
        acc[i] = zero16
        return carry
    lax.fori_loop(0, N, zero_row, 0)

    def per_query(j, carry):
        ii = nbr_v[j]
        d = dist_v[j]
        plsc.addupdate_scatter(acc, [ii, zero16i], d)
        plsc.addupdate_scatter(acc, [ii, one16i], one16f)
        gx = plsc.load_gather(d0xv, [ii])
        gy = plsc.load_gather(d0yv, [ii])
        gz = plsc.load_gather(d0zv, [ii])
        qspl = jnp.full((L,), base + j, I32)
        rx = plsc.load_gather(d0xv, [qspl])
        ry = plsc.load_gather(d0yv, [qspl])
        rz = plsc.load_gather(d0zv, [qspl])
        av[j] = gx * rx + gy * ry + gz * rz
        return carry

    lax.fori_loop(0, QPT, per_query, 0)
    pltpu.sync_copy(acc, densp_h.at[wid])
    pltpu.sync_copy(av, a_h.at[pl.ds(base, QPT)])


def _post_call(nbr, dist, d0x, d0y, d0z):
    f = pl.kernel(
        _post_body,
        out_type=(
            jax.ShapeDtypeStruct((NW, N, L), F32),
            jax.ShapeDtypeStruct((N, KNN), F32),
        ),
        mesh=_mesh(),
        compiler_params=_SC_PARAMS,
        scratch_types=[
            pltpu.VMEM((N,), F32), pltpu.VMEM((N,), F32), pltpu.VMEM((N,), F32),
            pltpu.VMEM((QPT, KNN), I32), pltpu.VMEM((QPT, KNN), F32),
            pltpu.VMEM((N, L), F32),
            pltpu.VMEM((QPT, KNN), F32),
        ],
    )
    return f(nbr, dist, d0x, d0y, d0z)


def _gather_body(snd_h, h_h, vec_h, hs_h, vs_h,
                 idxv, hrows0, vrows0, hrows1, vrows1, sem0, sem1):
    wid = _wid()
    pltpu.sync_copy(snd_h.at[pl.ds(wid * CPW, CPW)], idxv)
    ebase = wid * (CPW * 128)
    bufs = ((hrows0, vrows0, sem0), (hrows1, vrows1, sem1))

    def start(t, b):
        hr, vr, sem = bufs[b]
        pltpu.async_copy(h_h.at[idxv.at[t]], hr, sem)
        pltpu.async_copy(vec_h.at[idxv.at[t]], vr, sem)

    def finish(t, b):
        hr, vr, sem = bufs[b]
        pltpu.make_async_copy(h_h.at[idxv.at[t]], hr, sem).wait()
        pltpu.make_async_copy(vec_h.at[idxv.at[t]], vr, sem).wait()
        off = ebase + t * 128
        pltpu.sync_copy(hr, hs_h.at[pl.ds(off, 128)])
        pltpu.sync_copy(vr, vs_h.at[pl.ds(off, 128)])

    start(0, 0)

    def pair(g, carry):
        t0 = 2 * g
        start(t0 + 1, 1)
        finish(t0, 0)

        @pl.when(t0 + 2 < CPW)
        def _():
            start(t0 + 2, 0)

        finish(t0 + 1, 1)
        return carry

    lax.fori_loop(0, CPW // 2, pair, 0)


def _gather_call(snd2d, h, vec):
    f = pl.kernel(
        _gather_body,
        out_type=(
            jax.ShapeDtypeStruct((E, H), F32),
            jax.ShapeDtypeStruct((E, L), F32),
        ),
        mesh=_mesh(),
        compiler_params=_SC_PARAMS,
        scratch_types=[
            pltpu.VMEM((CPW, 128), I32),
            pltpu.VMEM((128, H), F32),
            pltpu.VMEM((128, L), F32),
            pltpu.VMEM((128, H), F32),
            pltpu.VMEM((128, L), F32),
            pltpu.SemaphoreType.DMA,
            pltpu.SemaphoreType.DMA,
        ],
    )
    return f(snd2d, h, vec)


def _scatter_body(snd_h, ef_h, t_h, z64_h, z16_h,
                  aggp_h, vecp_h,
                  acca, accb, idxv, rows64, rows16):
    cid = lax.axis_index("c")
    sid = lax.axis_index("s")
    wid = sid * NC + cid

    @pl.when(sid == 0)
    def _():
        pltpu.sync_copy(z64_h, acca)
        pltpu.sync_copy(z16_h, accb)

    plsc.subcore_barrier()
    pltpu.sync_copy(snd_h.at[pl.ds(wid * CPW, CPW)], idxv)

    def chunk(t, carry):
        off = wid * (CPW * 128) + t * 128
        isl = idxv.at[t]
        pltpu.sync_copy(ef_h.at[pl.ds(off, 128)], rows64)
        pltpu.sync_copy(rows64, acca.at[isl], add=True)
        pltpu.sync_copy(t_h.at[pl.ds(off, 128)], rows16)
        pltpu.sync_copy(rows16, accb.at[isl], add=True)
        return carry

    lax.fori_loop(0, CPW, chunk, 0)
    plsc.subcore_barrier()
    rows = N // NS
    pltpu.sync_copy(acca.at[pl.ds(sid * rows, rows)],
                    aggp_h.at[cid, pl.ds(sid * rows, rows)])
    pltpu.sync_copy(accb.at[pl.ds(sid * rows, rows)],
                    vecp_h.at[cid, pl.ds(sid * rows, rows)])


def _scatter_call(snd2d, ef, t, z64, z16):
    f = pl.kernel(
        _scatter_body,
        out_type=(
            jax.ShapeDtypeStruct((NC, N, H), F32),
            jax.ShapeDtypeStruct((NC, N, L), F32),
        ),
        mesh=_mesh(),
        compiler_params=_SC_PARAMS,
        scratch_types=[
            pltpu.VMEM_SHARED((N, H), F32),
            pltpu.VMEM_SHARED((N, L), F32),
            pltpu.VMEM((CPW, 128), I32),
            pltpu.VMEM((128, H), F32),
            pltpu.VMEM((128, L), F32),
        ],
    )
    return f(snd2d, ef, t, z64, z16)


def _bdot(a, b):
    return jnp.dot(a.astype(jnp.bfloat16), b.astype(jnp.bfloat16),
                   preferred_element_type=F32)


def _b16(x):
    return x.astype(jnp.bfloat16).astype(F32)


def _geom_body(dx_ref, dy_ref, dz_ref, dist_ref, dirx_ref, diry_ref, dirz_ref):
    dx = dx_ref[...]
    dy = dy_ref[...]
    dz = dz_ref[...]
    dist = jnp.sqrt(dx * dx + dy * dy + dz * dz)
    dist_ref[...] = dist
    den = dist + EPS
    dirx_ref[...] = dx / den
    diry_ref[...] = dy / den
    dirz_ref[...] = dz / den


def _geom_call(dx, dy, dz):
    return pl.pallas_call(
        _geom_body,
        out_shape=tuple(jax.ShapeDtypeStruct((N, KNN), F32) for _ in range(4)),
    )(dx, dy, dz)


def _densred_body(densp_ref, o_ref):
    @pl.when(pl.program_id(0) == 0)
    def _():
        o_ref[...] = jnp.zeros((N, L), F32)

    o_ref[...] += densp_ref[0]


def _densred_call(densp):
    return pl.pallas_call(
        _densred_body,
        grid=(NW,),
        in_specs=[pl.BlockSpec((1, N, L), lambda b: (b, 0, 0))],
        out_specs=pl.BlockSpec((N, L), lambda b: (0, 0)),
        out_shape=jax.ShapeDtypeStruct((N, L), F32),
    )(densp)


def _nodesetup_body(p16_ref, oht_ref, oh_ref, dens_ref, a_ref, emb_ref,
                    h0_ref, vec0_ref, ang_ref):
    p16 = p16_ref[...]
    dens2 = dens_ref[...]
    dsum = dens2[:, 0:1]
    dcnt = dens2[:, 1:2]
    density = jnp.where(dcnt > 0, dsum / jnp.maximum(dcnt, 1.0), 0.0)
    li = lax.broadcasted_iota(I32, (N, L), 1)
    pm = jnp.where(li < 3, p16, 0.0)
    loc = jnp.sqrt(jnp.sum(pm * pm, axis=1, keepdims=True))
    emb = emb_ref[...]
    h0_ref[...] = (_b16(loc) * _b16(emb[0:1]) + _b16(density) * _b16(emb[1:2])
                   + emb[2:3])
    seg = jnp.dot(oht_ref[...], p16, preferred_element_type=F32)
    bary = seg / seg[:, 3:4]
    vec0_ref[...] = p16 - jnp.dot(oh_ref[...], bary, preferred_element_type=F32)
    a = jnp.clip(a_ref[...], -1.0 + EPS, 1.0 - EPS)
    ang_ref[...] = jnp.arctan2(jnp.sqrt(jnp.maximum(1.0 - a * a, 0.0)), a)


def _nodesetup_call(p16, oht, oh, dens, a, emb):
    return pl.pallas_call(
        _nodesetup_body,
        out_shape=(
            jax.ShapeDtypeStruct((N, H), F32),
            jax.ShapeDtypeStruct((N, L), F32),
            jax.ShapeDtypeStruct((N, KNN), F32),
        ),
    )(p16, oht, oh, dens, a, emb)


def _edge1_body(hs_ref, h_ref, vs_ref, vec_ref, ea_ref,
                w1a_ref, w1b_ref, w1c_ref, wm_ref,
                e1_ref, vd_ref, st_ref):
    hs = hs_ref[...]
    hr = jnp.reshape(
        jnp.broadcast_to(h_ref[...][:, None, :], (QB, KNN, H)), (EB, H))
    vr = jnp.reshape(
        jnp.broadcast_to(vec_ref[...][:, None, :], (QB, KNN, L)), (EB, L))
    vd = vs_ref[...] - vr
    radial = jnp.sum(vd * vd, axis=1, keepdims=True)
    vd_ref[...] = vd / jnp.sqrt(radial) + EPS
    wm = wm_ref[...]
    e1 = (_bdot(hs, w1a_ref[...])
          + _bdot(hr, w1b_ref[...])
          + _bdot(ea_ref[...], w1c_ref[...])
          + _b16(radial) * _b16(wm[0:1]) + wm[1:2])
    e1_ref[...] = e1

    @pl.when(pl.program_id(0) == 0)
    def _():
        st_ref[...] = jnp.zeros((8, H), F32)

    s1 = jnp.sum(e1, axis=0)
    s2 = jnp.sum(e1 * e1, axis=0)
    st_ref[...] += jnp.concatenate(
        [s1[None, :], s2[None, :], jnp.zeros((6, H), F32)], axis=0)


def _edge1_call(hs, h, vs, vec, ea, w1a, w1b, w1c, wm):
    return pl.pallas_call(
        _edge1_body,
        grid=(NB,),
        in_specs=[
            pl.BlockSpec((EB, H), lambda b: (b, 0)),
            pl.BlockSpec((QB, H), lambda b: (b, 0)),
            pl.BlockSpec((EB, L), lambda b: (b, 0)),
            pl.BlockSpec((QB, L), lambda b: (b, 0)),
            pl.BlockSpec((EB, 8), lambda b: (b, 0)),
            pl.BlockSpec((H, H), lambda b: (0, 0)),
            pl.BlockSpec((H, H), lambda b: (0, 0)),
            pl.BlockSpec((8, H), lambda b: (0, 0)),
            pl.BlockSpec((8, H), lambda b: (0, 0)),
        ],
        out_specs=(
            pl.BlockSpec((EB, H), lambda b: (b, 0)),
            pl.BlockSpec((EB, L), lambda b: (b, 0)),
            pl.BlockSpec((8, H), lambda b: (0, 0)),
        ),
        out_shape=(
            jax.ShapeDtypeStruct((E, H), F32),
            jax.ShapeDtypeStruct((E, L), F32),
            jax.ShapeDtypeStruct((8, H), F32),
        ),
    )(hs, h, vs, vec, ea, w1a, w1b, w1c, wm)


def _edge2_body(e1_ref, st1_ref, ew2_ref, vw1_ref, wm_ref,
                ef_ref, st_ref):
    st1 = st1_ref[...]
    mean = st1[0:1] / E
    var = st1[1:2] / E - mean * mean
    wm = wm_ref[...]
    x = (e1_ref[...] - mean) / jnp.sqrt(var + 1e-5) * wm[0:1] + wm[1:2]
    m = jax.nn.silu(x)
    ef = jax.nn.silu(_bdot(m, ew2_ref[...]) + wm[2:3])
    ef_ref[...] = ef
    v1 = _bdot(ef, vw1_ref[...]) + wm[3:4]

    @pl.when(pl.program_id(0) == 0)
    def _():
        st_ref[...] = jnp.zeros((8, H), F32)

    s1 = jnp.sum(v1, axis=0)
    s2 = jnp.sum(v1 * v1, axis=0)
    st_ref[...] += jnp.concatenate(
        [s1[None, :], s2[None, :], jnp.zeros((6, H), F32)], axis=0)


def _edge2_call(e1, st1, ew2, vw1, wm):
    return pl.pallas_call(
        _edge2_body,
        grid=(NB,),
        in_specs=[
            pl.BlockSpec((EB, H), lambda b: (b, 0)),
            pl.BlockSpec((8, H), lambda b: (0, 0)),
            pl.BlockSpec((H, H), lambda b: (0, 0)),
            pl.BlockSpec((H, H), lambda b: (0, 0)),
            pl.BlockSpec((8, H), lambda b: (0, 0)),
        ],
        out_specs=(
            pl.BlockSpec((EB, H), lambda b: (b, 0)),
            pl.BlockSpec((8, H), lambda b: (0, 0)),
        ),
        out_shape=(
            jax.ShapeDtypeStruct((E, H), F32),
            jax.ShapeDtypeStruct((8, H), F32),
        ),
    )(e1, st1, ew2, vw1, wm)


def _edge3_body(ef_ref, vd_ref, st2_ref, vw1_ref, vw2_ref, wm_ref, t_ref):
    st2 = st2_ref[...]
    mean = st2[0:1] / E
    var = st2[1:2] / E - mean * mean
    wm = wm_ref[...]
    v1 = _bdot(ef_ref[...], vw1_ref[...]) + wm[2:3]
    m2 = jax.nn.silu((v1 - mean) / jnp.sqrt(var + 1e-5) * wm[0:1] + wm[1:2])
    vs8 = _bdot(m2, vw2_ref[...])
    vscale = vs8[:, 0:1]
    t_ref[...] = vd_ref[...] * vscale


def _edge3_call(ef, vd, st2, vw1, vw2, wm):
    return pl.pallas_call(
        _edge3_body,
        grid=(NB,),
        in_specs=[
            pl.BlockSpec((EB, H), lambda b: (b, 0)),
            pl.BlockSpec((EB, L), lambda b: (b, 0)),
            pl.BlockSpec((8, H), lambda b: (0, 0)),
            pl.BlockSpec((H, H), lambda b: (0, 0)),
            pl.BlockSpec((H, 8), lambda b: (0, 0)),
            pl.BlockSpec((8, H), lambda b: (0, 0)),
        ],
        out_specs=pl.BlockSpec((EB, L), lambda b: (b, 0)),
        out_shape=jax.ShapeDtypeStruct((E, L), F32),
    )(ef, vd, st2, vw1, vw2, wm)


def _node_body(h_ref, aggp_ref, vec_ref, vecp_ref,
               nw1a_ref, nw1b_ref, nw2_ref, wm_ref,
               ho_ref, vo_ref):
    h = h_ref[...]
    agg = aggp_ref[0] + aggp_ref[1]
    wm = wm_ref[...]
    nm1 = (_bdot(h, nw1a_ref[...])
           + _bdot(agg, nw1b_ref[...])
           + wm[0:1])
    mean = jnp.mean(nm1, axis=0, keepdims=True)
    var = jnp.mean((nm1 - mean) ** 2, axis=0, keepdims=True)
    nm = jax.nn.silu((nm1 - mean) / jnp.sqrt(var + 1e-5) * wm[1:2] + wm[2:3])
    ho_ref[...] = h + _bdot(nm, nw2_ref[...]) + wm[3:4]
    vadd = vecp_ref[0] + vecp_ref[1]
    li = lax.broadcasted_iota(I32, (N, L), 1)
    vo_ref[...] = vec_ref[...] + jnp.where(li < 3, vadd, 0.0)


def _node_call(h, aggp, vec, vecp, nw1a, nw1b, nw2, wm):
    return pl.pallas_call(
        _node_body,
        out_shape=(
            jax.ShapeDtypeStruct((N, H), F32),
            jax.ShapeDtypeStruct((N, L), F32),
        ),
    )(h, aggp, vec, vecp, nw1a, nw1b, nw2, wm)


def _proj_body(h_ref, w_ref, b_ref, o_ref):
    o_ref[...] = _bdot(h_ref[...], w_ref[...]) + b_ref[0:1]


def _proj_call(h, w, b8):
    return pl.pallas_call(
        _proj_body,
        out_shape=jax.ShapeDtypeStruct((N, C_DIM), F32),
    )(h, w, b8)


def kernel(pc, node_tag, params):
    p = pc.reshape(N, 3).astype(F32)
    tag = node_tag.reshape(N).astype(I32)
    px, py, pz = p[:, 0], p[:, 1], p[:, 2]

    bounds = jnp.searchsorted(tag, jnp.arange(NTAG + 1, dtype=I32)).astype(I32)
    st = bounds[tag].astype(F32)
    en = bounds[tag + 1].astype(F32)

    nbr, dxm, dym, dzm = _knn_call(px, py, pz, st, en)

    dist, dirx, diry, dirz = _geom_call(dxm, dym, dzm)

    d0x = dirx[:N // KNN].reshape(N)
    d0y = diry[:N // KNN].reshape(N)
    d0z = dirz[:N // KNN].reshape(N)
    densp, a = _post_call(nbr, dist, d0x, d0y, d0z)

    p16 = jnp.concatenate(
        [p, jnp.ones((N, 1), F32), jnp.zeros((N, L - 4), F32)], axis=1)
    oh = (tag[:, None] == jnp.arange(NTAG, dtype=I32)[None, :]).astype(F32)
    oht = oh.T
    emb = jnp.concatenate([
        params['emb_W'],
        params['emb_b'][None, :],
        jnp.zeros((5, H), F32)], axis=0)
    dens = _densred_call(densp)
    h, vec, ang = _nodesetup_call(p16, oht, oh, dens, a, emb)

    ea8 = jnp.concatenate(
        [dist.reshape(E, 1), ang.reshape(E, 1), jnp.zeros((E, 6), F32)], axis=1)
    snd2d = nbr.reshape(E // 128, 128)
    z64 = jnp.zeros((N, H), F32)
    z16 = jnp.zeros((N, L), F32)

    for lp in params['layers']:
        w1a = lp['e_W1'][:H]
        w1b = lp['e_W1'][H:2 * H]
        w1c = jnp.concatenate(
            [lp['e_W1'][2 * H + 1:], jnp.zeros((6, H), F32)], axis=0)
        wm1 = jnp.concatenate([
            lp['e_W1'][2 * H:2 * H + 1],
            lp['e_b1'][None, :],
            jnp.zeros((6, H), F32)], axis=0)
        wm2 = jnp.concatenate([
            lp['e_nw'][None, :], lp['e_nb'][None, :],
            lp['e_b2'][None, :], lp['v_b1'][None, :],
            jnp.zeros((4, H), F32)], axis=0)
        wm3 = jnp.concatenate([
            lp['v_nw'][None, :], lp['v_nb'][None, :],
            lp['v_b1'][None, :],
            jnp.zeros((5, H), F32)], axis=0)
        vw2 = jnp.concatenate([lp['v_W2'], jnp.zeros((H, 7), F32)], axis=1)
        wmn = jnp.concatenate([
            lp['n_b1'][None, :], lp['n_nw'][None, :],
            lp['n_nb'][None, :], lp['n_b2'][None, :],
            jnp.zeros((4, H), F32)], axis=0)

        hs, vs = _gather_call(snd2d, h, vec)
        e1, vd, st1 = _edge1_call(hs, h, vs, vec, ea8, w1a, w1b, w1c, wm1)
        ef, st2 = _edge2_call(e1, st1, lp['e_W2'], lp['v_W1'], wm2)
        t = _edge3_call(ef, vd, st2, lp['v_W1'], vw2, wm3)
        aggp, vecp = _scatter_call(snd2d, ef, t, z64, z16)
        h, vec = _node_call(h, aggp, vec, vecp,
                            lp['n_W1'][:H], lp['n_W1'][H:], lp['n_W2'], wmn)

    rob = jnp.concatenate(
        [params['ro_b'][None, :], jnp.zeros((7, C_DIM), F32)], axis=0)
    s_codes = _proj_call(h, params['ro_W'], rob).reshape(1, N, C_DIM)
    v_codes = vec[:, :3].reshape(1, N, 3)
    return v_codes, s_codes
